# baseline (speedup 1.0000x reference)
import functools

import jax
import jax.numpy as jnp
from jax import lax
from jax.experimental import pallas as pl
from jax.experimental.pallas import tpu as pltpu
from jax.experimental.pallas import tpu_sc as plsc

N_TOTAL = 64 * 512 * 512
NC = 2
NS = 16
L = 16
NW = NC * NS
PER_W = N_TOTAL // NW
CHUNK = 16384
N_CHUNKS = PER_W // CHUNK
VECS = CHUNK // L
UNROLL = 8

_mesh = plsc.VectorSubcoreMesh(core_axis_name="c", subcore_axis_name="s")


@functools.partial(
    pl.kernel,
    mesh=_mesh,
    out_type=jax.ShapeDtypeStruct((NW, 3 * L), jnp.float32),
    scratch_types=[
        pltpu.VMEM((CHUNK,), jnp.float32),
        pltpu.VMEM((CHUNK,), jnp.float32),
        pltpu.VMEM((3 * L,), jnp.float32),
    ],
)
def _wmse_partials(x_hbm, t_hbm, out_hbm, xbuf, tbuf, obuf):
    wid = lax.axis_index("s") * NC + lax.axis_index("c")
    base = wid * PER_W
    zero = jnp.zeros((L,), jnp.float32)
    one = jnp.ones((L,), jnp.float32)

    def chunk_body(ci, accs):
        off = base + ci * CHUNK
        pltpu.sync_copy(x_hbm.at[pl.ds(off, CHUNK)], xbuf)
        pltpu.sync_copy(t_hbm.at[pl.ds(off, CHUNK)], tbuf)

        def vec_body(vi, accs):
            fs, us, fc = accs
            for j in range(UNROLL):
                o = (vi * UNROLL + j) * L
                x = xbuf[pl.ds(o, L)]
                t = tbuf[pl.ds(o, L)]
                d = x - t
                sq = d * d
                m = t > zero
                fs = fs + jnp.where(m, sq, zero)
                us = us + jnp.where(m, zero, sq)
                fc = fc + jnp.where(m, one, zero)
            return (fs, us, fc)

        return lax.fori_loop(0, VECS // UNROLL, vec_body, accs)

    fs, us, fc = lax.fori_loop(0, N_CHUNKS, chunk_body, (zero, zero, zero))
    obuf[pl.ds(0, L)] = fs
    obuf[pl.ds(L, L)] = us
    obuf[pl.ds(2 * L, L)] = fc
    pltpu.sync_copy(obuf, out_hbm.at[wid])


def kernel(inputs, targets):
    x = inputs.reshape(-1)
    t = targets.reshape(-1)
    p = _wmse_partials(x, t).reshape(NW, 3, L)
    fs = jnp.sum(p[:, 0, :])
    us = jnp.sum(p[:, 1, :])
    fc = jnp.sum(p[:, 2, :])
    uc = jnp.float32(N_TOTAL) - fc
    flood = jnp.where(fc > 0, fs / jnp.maximum(fc, 1.0), 0.0)
    unflood = jnp.where(uc > 0, us / jnp.maximum(uc, 1.0), 0.0)
    loss = 20.0 * flood + unflood
    return (loss, flood, unflood)


# speedup vs baseline: 1.1922x; 1.1922x over previous
import functools

import jax
import jax.numpy as jnp
from jax import lax
from jax.experimental import pallas as pl
from jax.experimental.pallas import tpu as pltpu
from jax.experimental.pallas import tpu_sc as plsc

N_TOTAL = 64 * 512 * 512
NC = 2
NS = 16
L = 16
NW = NC * NS
PER_W = N_TOTAL // NW
CHUNK = 16384
N_CHUNKS = PER_W // CHUNK
VECS = CHUNK // L
UNROLL = 8

_mesh = plsc.VectorSubcoreMesh(core_axis_name="c", subcore_axis_name="s")


@functools.partial(
    pl.kernel,
    mesh=_mesh,
    out_type=jax.ShapeDtypeStruct((NW, 3 * L), jnp.float32),
    scratch_types=[
        pltpu.VMEM((2, CHUNK), jnp.float32),
        pltpu.VMEM((2, CHUNK), jnp.float32),
        pltpu.VMEM((3 * L,), jnp.float32),
        pltpu.SemaphoreType.DMA,
        pltpu.SemaphoreType.DMA,
    ],
)
def _wmse_partials(x_hbm, t_hbm, out_hbm, xbuf, tbuf, obuf, sem0, sem1):
    sems = (sem0, sem1)
    wid = lax.axis_index("s") * NC + lax.axis_index("c")
    base = wid * PER_W
    zero = jnp.zeros((L,), jnp.float32)
    one = jnp.ones((L,), jnp.float32)

    def start(ci, b):
        off = base + ci * CHUNK
        pltpu.async_copy(x_hbm.at[pl.ds(off, CHUNK)], xbuf.at[b], sems[b])
        pltpu.async_copy(t_hbm.at[pl.ds(off, CHUNK)], tbuf.at[b], sems[b])

    def wait(b):
        pltpu.make_async_copy(x_hbm.at[pl.ds(0, CHUNK)], xbuf.at[b], sems[b]).wait()
        pltpu.make_async_copy(t_hbm.at[pl.ds(0, CHUNK)], tbuf.at[b], sems[b]).wait()

    start(0, 0)
    start(1, 1)

    def pair_body(i, accs):
        for b in range(2):
            ci = 2 * i + b
            wait(b)
            xb = xbuf.at[b]
            tb = tbuf.at[b]

            def vec_body(vi, accs):
                fs, ts, fc = accs
                for j in range(UNROLL):
                    o = (vi * UNROLL + j) * L
                    x = xb[pl.ds(o, L)]
                    t = tb[pl.ds(o, L)]
                    d = x - t
                    sq = d * d
                    m = t > zero
                    fs = fs + jnp.where(m, sq, zero)
                    ts = ts + sq
                    fc = fc + jnp.where(m, one, zero)
                return (fs, ts, fc)

            accs = lax.fori_loop(0, VECS // UNROLL, vec_body, accs)

            @pl.when(ci + 2 < N_CHUNKS)
            def _():
                start(ci + 2, b)

        return accs

    fs, ts, fc = lax.fori_loop(0, N_CHUNKS // 2, pair_body, (zero, zero, zero))
    obuf[pl.ds(0, L)] = fs
    obuf[pl.ds(L, L)] = ts - fs
    obuf[pl.ds(2 * L, L)] = fc
    pltpu.sync_copy(obuf, out_hbm.at[wid])


def kernel(inputs, targets):
    x = inputs.reshape(-1)
    t = targets.reshape(-1)
    p = _wmse_partials(x, t).reshape(NW, 3, L)
    fs = jnp.sum(p[:, 0, :])
    us = jnp.sum(p[:, 1, :])
    fc = jnp.sum(p[:, 2, :])
    uc = jnp.float32(N_TOTAL) - fc
    flood = jnp.where(fc > 0, fs / jnp.maximum(fc, 1.0), 0.0)
    unflood = jnp.where(uc > 0, us / jnp.maximum(uc, 1.0), 0.0)
    loss = 20.0 * flood + unflood
    return (loss, flood, unflood)


# speedup vs baseline: 1.6219x; 1.3604x over previous
import functools

import jax
import jax.numpy as jnp
from jax import lax
from jax.experimental import pallas as pl
from jax.experimental.pallas import tpu as pltpu
from jax.experimental.pallas import tpu_sc as plsc

ROWS = 64 * 512
COLS = 512
N_TOTAL = ROWS * COLS
NC = 2
NS = 16
L = 16
NW = NC * NS
ROWS_W = ROWS // NW
CR = 16
N_CHUNKS = ROWS_W // CR
NACC = 4

_mesh = plsc.VectorSubcoreMesh(core_axis_name="c", subcore_axis_name="s")


@functools.partial(
    pl.kernel,
    mesh=_mesh,
    out_type=jax.ShapeDtypeStruct((NW, 3 * L), jnp.float32),
    scratch_types=[
        pltpu.VMEM((2, CR, COLS), jnp.float32),
        pltpu.VMEM((2, CR, COLS), jnp.float32),
        pltpu.VMEM((3 * L,), jnp.float32),
        pltpu.SemaphoreType.DMA,
        pltpu.SemaphoreType.DMA,
    ],
    compiler_params=pltpu.CompilerParams(
        use_tc_tiling_on_sc=True, needs_layout_passes=False
    ),
)
def _wmse_partials(x_hbm, t_hbm, out_hbm, xbuf, tbuf, obuf, sem0, sem1):
    sems = (sem0, sem1)
    wid = lax.axis_index("s") * NC + lax.axis_index("c")
    row0 = wid * ROWS_W
    zero = jnp.zeros((L,), jnp.float32)
    izero = jnp.zeros((L,), jnp.int32)

    def start(ci, b):
        r = row0 + ci * CR
        pltpu.async_copy(x_hbm.at[pl.ds(r, CR), :], xbuf.at[b], sems[b])
        pltpu.async_copy(t_hbm.at[pl.ds(r, CR), :], tbuf.at[b], sems[b])

    def wait(b):
        pltpu.make_async_copy(x_hbm.at[pl.ds(0, CR), :], xbuf.at[b], sems[b]).wait()
        pltpu.make_async_copy(t_hbm.at[pl.ds(0, CR), :], tbuf.at[b], sems[b]).wait()

    start(0, 0)
    start(1, 1)

    def pair_body(i, accs):
        for b in range(2):
            ci = 2 * i + b
            wait(b)
            xb = xbuf.at[b]
            tb = tbuf.at[b]
            fs, ts, fc = accs
            fs, ts, fc = list(fs), list(ts), list(fc)
            k = 0
            for r in range(CR):
                for g in range(COLS // L):
                    a = k % NACC
                    k += 1
                    x = xb[r, pl.ds(g * L, L)]
                    t = tb[r, pl.ds(g * L, L)]
                    d = x - t
                    sq = d * d
                    m = t > zero
                    fs[a] = fs[a] + jnp.where(m, sq, zero)
                    ts[a] = ts[a] + sq
                    fc[a] = fc[a] + plsc.all_reduce_population_count(m)
            accs = (tuple(fs), tuple(ts), tuple(fc))

            @pl.when(ci + 2 < N_CHUNKS)
            def _():
                start(ci + 2, b)

        return accs

    zf = (zero,) * NACC
    zi = (izero,) * NACC
    fs, ts, fc = lax.fori_loop(0, N_CHUNKS // 2, pair_body, (zf, zf, zi))
    fsum = fs[0] + fs[1] + fs[2] + fs[3]
    tsum = ts[0] + ts[1] + ts[2] + ts[3]
    csum = fc[0] + fc[1] + fc[2] + fc[3]
    obuf[pl.ds(0, L)] = fsum
    obuf[pl.ds(L, L)] = tsum - fsum
    obuf[pl.ds(2 * L, L)] = plsc.bitcast(csum, jnp.float32)
    pltpu.sync_copy(obuf, out_hbm.at[wid])


def kernel(inputs, targets):
    x = inputs.reshape(ROWS, COLS)
    t = targets.reshape(ROWS, COLS)
    p = _wmse_partials(x, t).reshape(NW, 3, L)
    fs = jnp.sum(p[:, 0, :])
    us = jnp.sum(p[:, 1, :])
    fc = jnp.sum(lax.bitcast_convert_type(p[:, 2, 0], jnp.int32)).astype(jnp.float32)
    uc = jnp.float32(N_TOTAL) - fc
    flood = jnp.where(fc > 0, fs / jnp.maximum(fc, 1.0), 0.0)
    unflood = jnp.where(uc > 0, us / jnp.maximum(uc, 1.0), 0.0)
    loss = 20.0 * flood + unflood
    return (loss, flood, unflood)


# speedup vs baseline: 3.2396x; 1.9974x over previous
import functools

import jax
import jax.numpy as jnp
from jax import lax
from jax.experimental import pallas as pl
from jax.experimental.pallas import tpu as pltpu
from jax.experimental.pallas import tpu_sc as plsc

ROWS = 64 * 512
COLS = 512
N_TOTAL = ROWS * COLS
NC = 2
NS = 16
L = 16
NW = NC * NS

ROWS_SC = 10240
ROWS_TC = ROWS - ROWS_SC
ROWS_W = ROWS_SC // NW
CR = 16
N_CHUNKS = ROWS_W // CR
NACC = 4

BR = 512
G = ROWS_TC // BR

_mesh = plsc.VectorSubcoreMesh(core_axis_name="c", subcore_axis_name="s")


@functools.partial(
    pl.kernel,
    mesh=_mesh,
    out_type=jax.ShapeDtypeStruct((NW, 3 * L), jnp.float32),
    scratch_types=[
        pltpu.VMEM((2, CR, COLS), jnp.float32),
        pltpu.VMEM((2, CR, COLS), jnp.float32),
        pltpu.VMEM((3 * L,), jnp.float32),
        pltpu.SemaphoreType.DMA,
        pltpu.SemaphoreType.DMA,
    ],
    compiler_params=pltpu.CompilerParams(
        use_tc_tiling_on_sc=True, needs_layout_passes=False
    ),
)
def _wmse_sc(x_hbm, t_hbm, out_hbm, xbuf, tbuf, obuf, sem0, sem1):
    sems = (sem0, sem1)
    wid = lax.axis_index("s") * NC + lax.axis_index("c")
    row0 = wid * ROWS_W
    zero = jnp.zeros((L,), jnp.float32)
    izero = jnp.zeros((L,), jnp.int32)

    def start(ci, b):
        r = row0 + ci * CR
        pltpu.async_copy(x_hbm.at[pl.ds(r, CR), :], xbuf.at[b], sems[b])
        pltpu.async_copy(t_hbm.at[pl.ds(r, CR), :], tbuf.at[b], sems[b])

    def wait(b):
        pltpu.make_async_copy(x_hbm.at[pl.ds(0, CR), :], xbuf.at[b], sems[b]).wait()
        pltpu.make_async_copy(t_hbm.at[pl.ds(0, CR), :], tbuf.at[b], sems[b]).wait()

    start(0, 0)
    start(1, 1)

    def pair_body(i, accs):
        for b in range(2):
            ci = 2 * i + b
            wait(b)
            xb = xbuf.at[b]
            tb = tbuf.at[b]
            fs, ts, fc = accs
            fs, ts, fc = list(fs), list(ts), list(fc)
            k = 0
            for r in range(CR):
                for g in range(COLS // L):
                    a = k % NACC
                    k += 1
                    x = xb[r, pl.ds(g * L, L)]
                    t = tb[r, pl.ds(g * L, L)]
                    d = x - t
                    sq = d * d
                    m = t > zero
                    fs[a] = fs[a] + jnp.where(m, sq, zero)
                    ts[a] = ts[a] + sq
                    fc[a] = fc[a] + plsc.all_reduce_population_count(m)
            accs = (tuple(fs), tuple(ts), tuple(fc))

            @pl.when(ci + 2 < N_CHUNKS)
            def _():
                start(ci + 2, b)

        return accs

    zf = (zero,) * NACC
    zi = (izero,) * NACC
    fs, ts, fc = lax.fori_loop(0, N_CHUNKS // 2, pair_body, (zf, zf, zi))
    fsum = fs[0] + fs[1] + fs[2] + fs[3]
    tsum = ts[0] + ts[1] + ts[2] + ts[3]
    csum = fc[0] + fc[1] + fc[2] + fc[3]
    obuf[pl.ds(0, L)] = fsum
    obuf[pl.ds(L, L)] = tsum - fsum
    obuf[pl.ds(2 * L, L)] = plsc.bitcast(csum, jnp.float32)
    pltpu.sync_copy(obuf, out_hbm.at[wid])


def _wmse_tc_body(x_ref, t_ref, o_ref):
    x = x_ref[...]
    t = t_ref[...]
    d = x - t
    sq = d * d
    m = t > 0.0
    fs = jnp.sum(jnp.where(m, sq, 0.0))
    ts = jnp.sum(sq)
    fc = jnp.sum(jnp.where(m, 1.0, 0.0))
    o_ref[0, 0, 0] = fs
    o_ref[0, 0, 1] = ts - fs
    o_ref[0, 0, 2] = fc


_TC_OFF = ROWS_SC // BR

_wmse_tc = pl.pallas_call(
    _wmse_tc_body,
    grid=(G,),
    in_specs=[
        pl.BlockSpec((BR, COLS), lambda i: (i + _TC_OFF, 0)),
        pl.BlockSpec((BR, COLS), lambda i: (i + _TC_OFF, 0)),
    ],
    out_specs=pl.BlockSpec((1, 1, 3), lambda i: (i, 0, 0), memory_space=pltpu.SMEM),
    out_shape=jax.ShapeDtypeStruct((G, 1, 3), jnp.float32),
)


def kernel(inputs, targets):
    x = inputs.reshape(ROWS, COLS)
    t = targets.reshape(ROWS, COLS)

    p_sc = _wmse_sc(x, t).reshape(NW, 3, L)
    p_tc = _wmse_tc(x, t)

    fs = jnp.sum(p_sc[:, 0, :]) + jnp.sum(p_tc[:, 0, 0])
    us = jnp.sum(p_sc[:, 1, :]) + jnp.sum(p_tc[:, 0, 1])
    fc = (
        jnp.sum(lax.bitcast_convert_type(p_sc[:, 2, 0], jnp.int32)).astype(jnp.float32)
        + jnp.sum(p_tc[:, 0, 2])
    )
    uc = jnp.float32(N_TOTAL) - fc
    flood = jnp.where(fc > 0, fs / jnp.maximum(fc, 1.0), 0.0)
    unflood = jnp.where(uc > 0, us / jnp.maximum(uc, 1.0), 0.0)
    loss = 20.0 * flood + unflood
    return (loss, flood, unflood)


# speedup vs baseline: 3.3354x; 1.0296x over previous
import functools

import jax
import jax.numpy as jnp
from jax import lax
from jax.experimental import pallas as pl
from jax.experimental.pallas import tpu as pltpu
from jax.experimental.pallas import tpu_sc as plsc

ROWS = 64 * 512
COLS = 512
N_TOTAL = ROWS * COLS
NC = 2
NS = 16
L = 16
NW = NC * NS

ROWS_SC = 10240
ROWS_TC = ROWS - ROWS_SC
ROWS_W = ROWS_SC // NW
CR = 16
N_CHUNKS = ROWS_W // CR
NACC = 4

BR = 512
G = ROWS_TC // BR

_mesh = plsc.VectorSubcoreMesh(core_axis_name="c", subcore_axis_name="s")


@functools.partial(
    pl.kernel,
    mesh=_mesh,
    out_type=jax.ShapeDtypeStruct((NW, 3 * L), jnp.float32),
    scratch_types=[
        pltpu.VMEM((2, CR, COLS), jnp.float32),
        pltpu.VMEM((2, CR, COLS), jnp.float32),
        pltpu.VMEM((3 * L,), jnp.float32),
        pltpu.SemaphoreType.DMA,
        pltpu.SemaphoreType.DMA,
    ],
    compiler_params=pltpu.CompilerParams(
        use_tc_tiling_on_sc=True,
        needs_layout_passes=False,
        skip_device_barrier=True,
    ),
)
def _wmse_sc(x_hbm, t_hbm, out_hbm, xbuf, tbuf, obuf, sem0, sem1):
    sems = (sem0, sem1)
    wid = lax.axis_index("s") * NC + lax.axis_index("c")
    row0 = wid * ROWS_W
    zero = jnp.zeros((L,), jnp.float32)
    izero = jnp.zeros((L,), jnp.int32)

    def start(ci, b):
        r = row0 + ci * CR
        pltpu.async_copy(x_hbm.at[pl.ds(r, CR), :], xbuf.at[b], sems[b])
        pltpu.async_copy(t_hbm.at[pl.ds(r, CR), :], tbuf.at[b], sems[b])

    def wait(b):
        pltpu.make_async_copy(x_hbm.at[pl.ds(0, CR), :], xbuf.at[b], sems[b]).wait()
        pltpu.make_async_copy(t_hbm.at[pl.ds(0, CR), :], tbuf.at[b], sems[b]).wait()

    start(0, 0)
    start(1, 1)

    def pair_body(i, accs):
        for b in range(2):
            ci = 2 * i + b
            wait(b)
            xb = xbuf.at[b]
            tb = tbuf.at[b]
            fs, ts, fc = accs
            fs, ts, fc = list(fs), list(ts), list(fc)
            k = 0
            for r in range(CR):
                for g in range(COLS // L):
                    a = k % NACC
                    k += 1
                    x = xb[r, pl.ds(g * L, L)]
                    t = tb[r, pl.ds(g * L, L)]
                    d = x - t
                    sq = d * d
                    m = t > zero
                    fs[a] = fs[a] + jnp.where(m, sq, zero)
                    ts[a] = ts[a] + sq
                    fc[a] = fc[a] + plsc.all_reduce_population_count(m)
            accs = (tuple(fs), tuple(ts), tuple(fc))

            @pl.when(ci + 2 < N_CHUNKS)
            def _():
                start(ci + 2, b)

        return accs

    zf = (zero,) * NACC
    zi = (izero,) * NACC
    fs, ts, fc = lax.fori_loop(0, N_CHUNKS // 2, pair_body, (zf, zf, zi))
    fsum = fs[0] + fs[1] + fs[2] + fs[3]
    tsum = ts[0] + ts[1] + ts[2] + ts[3]
    csum = fc[0] + fc[1] + fc[2] + fc[3]
    obuf[pl.ds(0, L)] = fsum
    obuf[pl.ds(L, L)] = tsum - fsum
    obuf[pl.ds(2 * L, L)] = plsc.bitcast(csum, jnp.float32)
    pltpu.sync_copy(obuf, out_hbm.at[wid])


def _wmse_tc_body(x_ref, t_ref, o_ref):
    x = x_ref[...]
    t = t_ref[...]
    d = x - t
    sq = d * d
    m = t > 0.0
    fs = jnp.sum(jnp.where(m, sq, 0.0))
    ts = jnp.sum(sq)
    fc = jnp.sum(jnp.where(m, 1.0, 0.0))
    o_ref[0, 0, 0] = fs
    o_ref[0, 0, 1] = ts - fs
    o_ref[0, 0, 2] = fc


_TC_OFF = ROWS_SC // BR

_wmse_tc = pl.pallas_call(
    _wmse_tc_body,
    grid=(G,),
    in_specs=[
        pl.BlockSpec((BR, COLS), lambda i: (i + _TC_OFF, 0)),
        pl.BlockSpec((BR, COLS), lambda i: (i + _TC_OFF, 0)),
    ],
    out_specs=pl.BlockSpec((1, 1, 3), lambda i: (i, 0, 0), memory_space=pltpu.SMEM),
    out_shape=jax.ShapeDtypeStruct((G, 1, 3), jnp.float32),
)


def kernel(inputs, targets):
    x = inputs.reshape(ROWS, COLS)
    t = targets.reshape(ROWS, COLS)

    p_sc = _wmse_sc(x, t).reshape(NW, 3, L)
    p_tc = _wmse_tc(x, t)

    fs = jnp.sum(p_sc[:, 0, :]) + jnp.sum(p_tc[:, 0, 0])
    us = jnp.sum(p_sc[:, 1, :]) + jnp.sum(p_tc[:, 0, 1])
    fc = (
        jnp.sum(lax.bitcast_convert_type(p_sc[:, 2, 0], jnp.int32)).astype(jnp.float32)
        + jnp.sum(p_tc[:, 0, 2])
    )
    uc = jnp.float32(N_TOTAL) - fc
    flood = jnp.where(fc > 0, fs / jnp.maximum(fc, 1.0), 0.0)
    unflood = jnp.where(uc > 0, us / jnp.maximum(uc, 1.0), 0.0)
    loss = 20.0 * flood + unflood
    return (loss, flood, unflood)


# speedup vs baseline: 3.3652x; 1.0089x over previous
import functools

import jax
import jax.numpy as jnp
from jax import lax
from jax.experimental import pallas as pl
from jax.experimental.pallas import tpu as pltpu
from jax.experimental.pallas import tpu_sc as plsc

ROWS = 64 * 512
COLS = 512
N_TOTAL = ROWS * COLS
NC = 2
NS = 16
L = 16
NW = NC * NS

ROWS_SC = 10240
ROWS_TC = ROWS - ROWS_SC
ROWS_W = ROWS_SC // NW
CR = 8
N_CHUNKS = ROWS_W // CR
NBUF = 4
NACC = 4

BR = 512
G = ROWS_TC // BR

_mesh = plsc.VectorSubcoreMesh(core_axis_name="c", subcore_axis_name="s")


@functools.partial(
    pl.kernel,
    mesh=_mesh,
    out_type=jax.ShapeDtypeStruct((NW, 3 * L), jnp.float32),
    scratch_types=[
        pltpu.VMEM((NBUF, CR, COLS), jnp.float32),
        pltpu.VMEM((NBUF, CR, COLS), jnp.float32),
        pltpu.VMEM((3 * L,), jnp.float32),
        pltpu.SemaphoreType.DMA((NBUF,)),
    ],
    compiler_params=pltpu.CompilerParams(
        use_tc_tiling_on_sc=True,
        needs_layout_passes=False,
        skip_device_barrier=True,
    ),
)
def _wmse_sc(x_hbm, t_hbm, out_hbm, xbuf, tbuf, obuf, sems):
    wid = lax.axis_index("s") * NC + lax.axis_index("c")
    row0 = wid * ROWS_W
    zero = jnp.zeros((L,), jnp.float32)
    izero = jnp.zeros((L,), jnp.int32)

    def start(ci, b):
        r = row0 + ci * CR
        pltpu.async_copy(x_hbm.at[pl.ds(r, CR), :], xbuf.at[b], sems.at[b])
        pltpu.async_copy(t_hbm.at[pl.ds(r, CR), :], tbuf.at[b], sems.at[b])

    def wait(b):
        pltpu.make_async_copy(x_hbm.at[pl.ds(0, CR), :], xbuf.at[b], sems.at[b]).wait()
        pltpu.make_async_copy(t_hbm.at[pl.ds(0, CR), :], tbuf.at[b], sems.at[b]).wait()

    for b in range(NBUF):
        start(b, b)

    def chunk_body(ci, accs):
        b = lax.rem(ci, NBUF)
        wait(b)

        def row_body(r, accs):
            fs, ts, fc = accs
            fs, ts, fc = list(fs), list(ts), list(fc)
            for g in range(COLS // L):
                a = g % NACC
                x = xbuf[b, r, pl.ds(g * L, L)]
                t = tbuf[b, r, pl.ds(g * L, L)]
                d = x - t
                sq = d * d
                m = t > zero
                fs[a] = fs[a] + jnp.where(m, sq, zero)
                ts[a] = ts[a] + sq
                fc[a] = fc[a] + plsc.all_reduce_population_count(m)
            return (tuple(fs), tuple(ts), tuple(fc))

        accs = lax.fori_loop(0, CR, row_body, accs)

        @pl.when(ci + NBUF < N_CHUNKS)
        def _():
            start(ci + NBUF, b)

        return accs

    zf = (zero,) * NACC
    zi = (izero,) * NACC
    fs, ts, fc = lax.fori_loop(0, N_CHUNKS, chunk_body, (zf, zf, zi))
    fsum = fs[0] + fs[1] + fs[2] + fs[3]
    tsum = ts[0] + ts[1] + ts[2] + ts[3]
    csum = fc[0] + fc[1] + fc[2] + fc[3]
    obuf[pl.ds(0, L)] = fsum
    obuf[pl.ds(L, L)] = tsum - fsum
    obuf[pl.ds(2 * L, L)] = plsc.bitcast(csum, jnp.float32)
    pltpu.sync_copy(obuf, out_hbm.at[wid])


def _wmse_tc_body(x_ref, t_ref, o_ref):
    x = x_ref[...]
    t = t_ref[...]
    d = x - t
    sq = d * d
    m = t > 0.0
    fs = jnp.sum(jnp.where(m, sq, 0.0))
    ts = jnp.sum(sq)
    fc = jnp.sum(jnp.where(m, 1.0, 0.0))
    o_ref[0, 0, 0] = fs
    o_ref[0, 0, 1] = ts - fs
    o_ref[0, 0, 2] = fc


_TC_OFF = ROWS_SC // BR

_wmse_tc = pl.pallas_call(
    _wmse_tc_body,
    grid=(G,),
    in_specs=[
        pl.BlockSpec((BR, COLS), lambda i: (i + _TC_OFF, 0)),
        pl.BlockSpec((BR, COLS), lambda i: (i + _TC_OFF, 0)),
    ],
    out_specs=pl.BlockSpec((1, 1, 3), lambda i: (i, 0, 0), memory_space=pltpu.SMEM),
    out_shape=jax.ShapeDtypeStruct((G, 1, 3), jnp.float32),
)


def kernel(inputs, targets):
    x = inputs.reshape(ROWS, COLS)
    t = targets.reshape(ROWS, COLS)

    p_sc = _wmse_sc(x, t).reshape(NW, 3, L)
    p_tc = _wmse_tc(x, t)

    fs = jnp.sum(p_sc[:, 0, :]) + jnp.sum(p_tc[:, 0, 0])
    us = jnp.sum(p_sc[:, 1, :]) + jnp.sum(p_tc[:, 0, 1])
    fc = (
        jnp.sum(lax.bitcast_convert_type(p_sc[:, 2, 0], jnp.int32)).astype(jnp.float32)
        + jnp.sum(p_tc[:, 0, 2])
    )
    uc = jnp.float32(N_TOTAL) - fc
    flood = jnp.where(fc > 0, fs / jnp.maximum(fc, 1.0), 0.0)
    unflood = jnp.where(uc > 0, us / jnp.maximum(uc, 1.0), 0.0)
    loss = 20.0 * flood + unflood
    return (loss, flood, unflood)
